# baseline (speedup 1.0000x reference)
import jax
import jax.numpy as jnp
from jax import lax
from jax.experimental import pallas as pl

TOP_K = 48
NUM_RBF = 16
MAX_REL = 32

_N, _CA, _C, _O, _CB, _RIDX = 0, 3, 6, 9, 12, 15
_PAIRS = [(_CA, _CA), (_N, _N), (_C, _C), (_O, _O), (_CB, _CB),
          (_CA, _N), (_CA, _C), (_CA, _O), (_CA, _CB), (_N, _C),
          (_N, _O), (_N, _CB), (_CB, _C), (_CB, _O), (_O, _C),
          (_N, _CA), (_C, _CA), (_O, _CA), (_CB, _CA), (_C, _N),
          (_O, _N), (_CB, _N), (_C, _CB), (_O, _CB), (_C, _O)]


def _atoms_kernel(x_ref, ridx_ref, out_ref):
    x = x_ref[...]
    nx, ny, nz = x[:, 0:1], x[:, 1:2], x[:, 2:3]
    cax, cay, caz = x[:, 3:4], x[:, 4:5], x[:, 5:6]
    cx, cy, cz = x[:, 6:7], x[:, 7:8], x[:, 8:9]
    bx, by, bz = cax - nx, cay - ny, caz - nz
    ccx, ccy, ccz = cx - cax, cy - cay, cz - caz
    ax = by * ccz - bz * ccy
    ay = bz * ccx - bx * ccz
    az = bx * ccy - by * ccx
    cbx = -0.58273431 * ax + 0.56802827 * bx - 0.54067466 * ccx + cax
    cby = -0.58273431 * ay + 0.56802827 * by - 0.54067466 * ccy + cay
    cbz = -0.58273431 * az + 0.56802827 * bz - 0.54067466 * ccz + caz
    out_ref[:, 0:12] = x
    out_ref[:, 12:13] = cbx
    out_ref[:, 13:14] = cby
    out_ref[:, 14:15] = cbz
    out_ref[:, 15:16] = ridx_ref[...]


def _topk_kernel(carows_ref, caT_ref, mrow_ref, mcol_ref, out_ref):
    caT = caT_ref[0]
    cr = carows_ref[...]
    d2 = None
    for c in range(3):
        diff = cr[:, c:c + 1] - caT[c:c + 1, :]
        sq = diff * diff
        d2 = sq if d2 is None else d2 + sq
    m2 = mrow_ref[...] * mcol_ref[0]
    dist = m2 * jnp.sqrt(d2 + 1e-6)
    dmax = jnp.max(dist, axis=1, keepdims=True)
    vals = dist + (1.0 - m2) * dmax
    iota = lax.broadcasted_iota(jnp.int32, vals.shape, 1).astype(jnp.float32)
    L = vals.shape[1]
    cols = []
    for _ in range(TOP_K):
        mn = jnp.min(vals, axis=1, keepdims=True)
        sel = vals <= mn
        idxf = jnp.min(jnp.where(sel, iota, float(2 * L)), axis=1,
                       keepdims=True)
        cols.append(idxf)
        vals = jnp.where(iota == idxf, 3.0e38, vals)
    idx_all = jnp.concatenate(cols, axis=1)
    out_ref[...] = idx_all.astype(jnp.int32) + pl.program_id(0) * L


def _edge_kernel(eidx_ref, atoms_all_ref, rows_ref, wpeT_ref, bpe_ref,
                 mu_ref, wedgeT_ref, gam_ref, bet_ref, out_ref):
    atoms = atoms_all_ref[0]
    L = atoms.shape[0]
    T = rows_ref.shape[0]
    E = eidx_ref.shape[0]
    eloc = eidx_ref[...] - pl.program_id(0) * L
    iota_l = lax.broadcasted_iota(jnp.int32, (E, L), 1)
    oh = jnp.where(eloc == iota_l, 1.0, 0.0)
    nb = jnp.dot(oh, atoms, preferred_element_type=jnp.float32,
                 precision=lax.Precision.HIGHEST)
    ie = lax.broadcasted_iota(jnp.int32, (E, T), 0)
    ir = lax.broadcasted_iota(jnp.int32, (E, T), 1) * TOP_K
    rep = jnp.where((ie >= ir) & (ie < ir + TOP_K), 1.0, 0.0)
    center = jnp.dot(rep, rows_ref[...],
                     preferred_element_type=jnp.float32,
                     precision=lax.Precision.HIGHEST)
    ccol = [center[:, i:i + 1] for i in range(16)]
    ncol = [nb[:, i:i + 1] for i in range(16)]
    dcls = jnp.clip(ccol[_RIDX] - ncol[_RIDX] + float(MAX_REL),
                    0.0, float(2 * MAX_REL))
    iota66 = lax.broadcasted_iota(jnp.int32, (E, 2 * MAX_REL + 2),
                                  1).astype(jnp.float32)
    oh66 = jnp.where((iota66 > dcls - 0.5) & (iota66 < dcls + 0.5), 1.0, 0.0)
    pos = jnp.dot(oh66, wpeT_ref[...],
                  preferred_element_type=jnp.float32,
                  precision=lax.Precision.HIGHEST) + bpe_ref[...]
    mu = mu_ref[...]
    sigma = (22.0 - 2.0) / NUM_RBF
    parts = [pos]
    for ao, bo in _PAIRS:
        da = ccol[ao] - ncol[bo]
        db = ccol[ao + 1] - ncol[bo + 1]
        dc = ccol[ao + 2] - ncol[bo + 2]
        dist = jnp.sqrt(da * da + db * db + dc * dc + 1e-6)
        z = (dist - mu) / sigma
        parts.append(jnp.exp(-(z * z)))
    feats = jnp.concatenate(parts, axis=1)
    e_out = jnp.dot(feats, wedgeT_ref[...],
                    preferred_element_type=jnp.float32)
    m = jnp.mean(e_out, axis=1, keepdims=True)
    var = jnp.mean((e_out - m) ** 2, axis=1, keepdims=True)
    out_ref[...] = ((e_out - m) / jnp.sqrt(var + 1e-5)
                    * gam_ref[...] + bet_ref[...])


def kernel(X, mask, residue_idx, W_pe, b_pe, W_edge, ln_gamma, ln_beta):
    B, L = X.shape[0], X.shape[1]
    K = min(TOP_K, L)
    BL = B * L
    EF = W_edge.shape[0]

    x_flat = X.reshape(BL, 12)
    ridx_f = residue_idx.astype(jnp.float32).reshape(BL, 1)
    atoms = pl.pallas_call(
        _atoms_kernel,
        out_shape=jax.ShapeDtypeStruct((BL, 16), jnp.float32),
    )(x_flat, ridx_f)

    TT = 128
    nblk = L // TT
    ca_rows = X[:, :, 1, :].reshape(BL, 3)
    caT = jnp.transpose(X[:, :, 1, :], (0, 2, 1))
    mask_row = mask.reshape(BL, 1)
    mask_col = mask.reshape(B, 1, L)
    eidx = pl.pallas_call(
        _topk_kernel,
        grid=(B, nblk),
        in_specs=[
            pl.BlockSpec((TT, 3), lambda b, t: (b * nblk + t, 0)),
            pl.BlockSpec((1, 3, L), lambda b, t: (b, 0, 0)),
            pl.BlockSpec((TT, 1), lambda b, t: (b * nblk + t, 0)),
            pl.BlockSpec((1, 1, L), lambda b, t: (b, 0, 0)),
        ],
        out_specs=pl.BlockSpec((TT, K), lambda b, t: (b * nblk + t, 0)),
        out_shape=jax.ShapeDtypeStruct((BL, K), jnp.int32),
    )(ca_rows, caT, mask_row, mask_col)

    T = 16
    EB = T * K
    nblk2 = L // T
    eidx_flat = eidx.reshape(BL * K, 1)
    atoms3 = atoms.reshape(B, L, 16)
    wpeT = W_pe.T
    bpe2 = b_pe.reshape(1, -1)
    mu_row = jnp.linspace(2.0, 22.0, NUM_RBF).reshape(1, NUM_RBF)
    wedgeT = W_edge.T
    gam2 = ln_gamma.reshape(1, -1)
    bet2 = ln_beta.reshape(1, -1)
    full = lambda s: pl.BlockSpec(s, lambda b, t: tuple(0 for _ in s))
    e_flat = pl.pallas_call(
        _edge_kernel,
        grid=(B, nblk2),
        in_specs=[
            pl.BlockSpec((EB, 1), lambda b, t: (b * nblk2 + t, 0)),
            pl.BlockSpec((1, L, 16), lambda b, t: (b, 0, 0)),
            pl.BlockSpec((T, 16), lambda b, t: (b * nblk2 + t, 0)),
            full(wpeT.shape),
            full(bpe2.shape),
            full(mu_row.shape),
            full(wedgeT.shape),
            full(gam2.shape),
            full(bet2.shape),
        ],
        out_specs=pl.BlockSpec((EB, EF), lambda b, t: (b * nblk2 + t, 0)),
        out_shape=jax.ShapeDtypeStruct((BL * K, EF), jnp.float32),
    )(eidx_flat, atoms3, atoms, wpeT, bpe2, mu_row,
      wedgeT, gam2, bet2)
    return e_flat.reshape(B, L, K, EF)


# speedup vs baseline: 1.2535x; 1.2535x over previous
import functools

import jax
import jax.numpy as jnp
from jax import lax
from jax.experimental import pallas as pl
from jax.experimental.pallas import tpu as pltpu
from jax.experimental.pallas import tpu_sc as plsc

TOP_K = 48
NUM_RBF = 16
MAX_REL = 32

_N, _CA, _C, _O, _CB, _RIDX = 0, 3, 6, 9, 12, 15
_PAIRS = [(_CA, _CA), (_N, _N), (_C, _C), (_O, _O), (_CB, _CB),
          (_CA, _N), (_CA, _C), (_CA, _O), (_CA, _CB), (_N, _C),
          (_N, _O), (_N, _CB), (_CB, _C), (_CB, _O), (_O, _C),
          (_N, _CA), (_C, _CA), (_O, _CA), (_CB, _CA), (_C, _N),
          (_O, _N), (_CB, _N), (_C, _CB), (_O, _CB), (_C, _O)]


def _atoms_kernel(x_ref, ridx_ref, out_ref):
    x = x_ref[...]
    nx, ny, nz = x[:, 0:1], x[:, 1:2], x[:, 2:3]
    cax, cay, caz = x[:, 3:4], x[:, 4:5], x[:, 5:6]
    cx, cy, cz = x[:, 6:7], x[:, 7:8], x[:, 8:9]
    bx, by, bz = cax - nx, cay - ny, caz - nz
    ccx, ccy, ccz = cx - cax, cy - cay, cz - caz
    ax = by * ccz - bz * ccy
    ay = bz * ccx - bx * ccz
    az = bx * ccy - by * ccx
    cbx = -0.58273431 * ax + 0.56802827 * bx - 0.54067466 * ccx + cax
    cby = -0.58273431 * ay + 0.56802827 * by - 0.54067466 * ccy + cay
    cbz = -0.58273431 * az + 0.56802827 * bz - 0.54067466 * ccz + caz
    out_ref[:, 0:12] = x
    out_ref[:, 12:13] = cbx
    out_ref[:, 13:14] = cby
    out_ref[:, 14:15] = cbz
    out_ref[:, 15:16] = ridx_ref[...]


def _topk_kernel(carows_ref, caT_ref, mrow_ref, mcol_ref, out_ref):
    caT = caT_ref[0]
    cr = carows_ref[...]
    d2 = None
    for c in range(3):
        diff = cr[:, c:c + 1] - caT[c:c + 1, :]
        sq = diff * diff
        d2 = sq if d2 is None else d2 + sq
    m2 = mrow_ref[...] * mcol_ref[0]
    dist = m2 * jnp.sqrt(d2 + 1e-6)
    dmax = jnp.max(dist, axis=1, keepdims=True)
    vals = dist + (1.0 - m2) * dmax
    iota = lax.broadcasted_iota(jnp.int32, vals.shape, 1).astype(jnp.float32)
    L = vals.shape[1]
    cols = []
    for _ in range(TOP_K):
        mn = jnp.min(vals, axis=1, keepdims=True)
        sel = vals <= mn
        idxf = jnp.min(jnp.where(sel, iota, float(2 * L)), axis=1,
                       keepdims=True)
        cols.append(idxf)
        vals = jnp.where(iota == idxf, 3.0e38, vals)
    idx_all = jnp.concatenate(cols, axis=1)
    out_ref[...] = idx_all.astype(jnp.int32) + pl.program_id(0) * L


def _sc_gather(table, idx, D):
    Bn = idx.shape[0]
    info = plsc.get_sparse_core_info()
    NW = info.num_cores * info.num_subcores
    b_per_w = Bn // NW

    @functools.partial(
        pl.kernel,
        mesh=plsc.VectorSubcoreMesh(core_axis_name="c", subcore_axis_name="s"),
        compiler_params=pltpu.CompilerParams(use_tc_tiling_on_sc=False),
        out_type=jax.ShapeDtypeStruct((Bn, D), jnp.float32),
        scratch_types=[
            pltpu.VMEM((b_per_w,), jnp.int32),
            pltpu.VMEM((b_per_w, D), jnp.float32),
            pltpu.SemaphoreType.DMA,
        ],
    )
    def k(table_hbm, idx_hbm, out_hbm, idx_v, rows_v, sem):
        wid = lax.axis_index("s") * info.num_cores + lax.axis_index("c")
        base = wid * b_per_w
        pltpu.sync_copy(idx_hbm.at[pl.ds(base, b_per_w)], idx_v)
        pltpu.async_copy(table_hbm.at[idx_v], rows_v, sem).wait()
        pltpu.sync_copy(rows_v, out_hbm.at[pl.ds(base, b_per_w)])

    return k(table, idx)


def _edge_kernel(nb_ref, rows_ref, wpeT_ref, bpe_ref,
                 mu_ref, wedgeT_ref, gam_ref, bet_ref, out_ref):
    T = rows_ref.shape[0]
    E = nb_ref.shape[0]
    nb = nb_ref[...]
    ie = lax.broadcasted_iota(jnp.int32, (E, T), 0)
    ir = lax.broadcasted_iota(jnp.int32, (E, T), 1) * TOP_K
    rep = jnp.where((ie >= ir) & (ie < ir + TOP_K), 1.0, 0.0)
    center = jnp.dot(rep, rows_ref[...],
                     preferred_element_type=jnp.float32,
                     precision=lax.Precision.HIGHEST)
    ccol = [center[:, i:i + 1] for i in range(16)]
    ncol = [nb[:, i:i + 1] for i in range(16)]
    dcls = jnp.clip(ccol[_RIDX] - ncol[_RIDX] + float(MAX_REL),
                    0.0, float(2 * MAX_REL))
    iota66 = lax.broadcasted_iota(jnp.int32, (E, 2 * MAX_REL + 2),
                                  1).astype(jnp.float32)
    oh66 = jnp.where((iota66 > dcls - 0.5) & (iota66 < dcls + 0.5), 1.0, 0.0)
    pos = jnp.dot(oh66, wpeT_ref[...],
                  preferred_element_type=jnp.float32,
                  precision=lax.Precision.HIGHEST) + bpe_ref[...]
    mu = mu_ref[...]
    sigma = (22.0 - 2.0) / NUM_RBF
    parts = [pos]
    for ao, bo in _PAIRS:
        da = ccol[ao] - ncol[bo]
        db = ccol[ao + 1] - ncol[bo + 1]
        dc = ccol[ao + 2] - ncol[bo + 2]
        dist = jnp.sqrt(da * da + db * db + dc * dc + 1e-6)
        z = (dist - mu) / sigma
        parts.append(jnp.exp(-(z * z)))
    feats = jnp.concatenate(parts, axis=1)
    e_out = jnp.dot(feats, wedgeT_ref[...],
                    preferred_element_type=jnp.float32)
    m = jnp.mean(e_out, axis=1, keepdims=True)
    var = jnp.mean((e_out - m) ** 2, axis=1, keepdims=True)
    out_ref[...] = ((e_out - m) / jnp.sqrt(var + 1e-5)
                    * gam_ref[...] + bet_ref[...])


def kernel(X, mask, residue_idx, W_pe, b_pe, W_edge, ln_gamma, ln_beta):
    B, L = X.shape[0], X.shape[1]
    K = min(TOP_K, L)
    BL = B * L
    EF = W_edge.shape[0]

    x_flat = X.reshape(BL, 12)
    ridx_f = residue_idx.astype(jnp.float32).reshape(BL, 1)
    atoms = pl.pallas_call(
        _atoms_kernel,
        out_shape=jax.ShapeDtypeStruct((BL, 16), jnp.float32),
    )(x_flat, ridx_f)

    TT = 128
    nblk = L // TT
    ca_rows = X[:, :, 1, :].reshape(BL, 3)
    caT = jnp.transpose(X[:, :, 1, :], (0, 2, 1))
    mask_row = mask.reshape(BL, 1)
    mask_col = mask.reshape(B, 1, L)
    eidx = pl.pallas_call(
        _topk_kernel,
        grid=(B, nblk),
        in_specs=[
            pl.BlockSpec((TT, 3), lambda b, t: (b * nblk + t, 0)),
            pl.BlockSpec((1, 3, L), lambda b, t: (b, 0, 0)),
            pl.BlockSpec((TT, 1), lambda b, t: (b * nblk + t, 0)),
            pl.BlockSpec((1, 1, L), lambda b, t: (b, 0, 0)),
        ],
        out_specs=pl.BlockSpec((TT, K), lambda b, t: (b * nblk + t, 0)),
        out_shape=jax.ShapeDtypeStruct((BL, K), jnp.int32),
    )(ca_rows, caT, mask_row, mask_col)

    nb_flat = _sc_gather(atoms, eidx.reshape(BL * K), 16)

    T = 16
    EB = T * K
    nblk2 = L // T
    wpeT = W_pe.T
    bpe2 = b_pe.reshape(1, -1)
    mu_row = jnp.linspace(2.0, 22.0, NUM_RBF).reshape(1, NUM_RBF)
    wedgeT = W_edge.T
    gam2 = ln_gamma.reshape(1, -1)
    bet2 = ln_beta.reshape(1, -1)
    full = lambda s: pl.BlockSpec(s, lambda b, t: tuple(0 for _ in s))
    e_flat = pl.pallas_call(
        _edge_kernel,
        grid=(B, nblk2),
        in_specs=[
            pl.BlockSpec((EB, 16), lambda b, t: (b * nblk2 + t, 0)),
            pl.BlockSpec((T, 16), lambda b, t: (b * nblk2 + t, 0)),
            full(wpeT.shape),
            full(bpe2.shape),
            full(mu_row.shape),
            full(wedgeT.shape),
            full(gam2.shape),
            full(bet2.shape),
        ],
        out_specs=pl.BlockSpec((EB, EF), lambda b, t: (b * nblk2 + t, 0)),
        out_shape=jax.ShapeDtypeStruct((BL * K, EF), jnp.float32),
    )(nb_flat, atoms, wpeT, bpe2, mu_row,
      wedgeT, gam2, bet2)
    return e_flat.reshape(B, L, K, EF)


# speedup vs baseline: 2.0594x; 1.6429x over previous
import functools

import numpy as np

import jax
import jax.numpy as jnp
from jax import lax
from jax.experimental import pallas as pl
from jax.experimental.pallas import tpu as pltpu
from jax.experimental.pallas import tpu_sc as plsc

TOP_K = 48
NUM_RBF = 16
MAX_REL = 32

_N, _CA, _C, _O, _CB, _RIDX = 0, 3, 6, 9, 12, 15
_PAIRS = [(_CA, _CA), (_N, _N), (_C, _C), (_O, _O), (_CB, _CB),
          (_CA, _N), (_CA, _C), (_CA, _O), (_CA, _CB), (_N, _C),
          (_N, _O), (_N, _CB), (_CB, _C), (_CB, _O), (_O, _C),
          (_N, _CA), (_C, _CA), (_O, _CA), (_CB, _CA), (_C, _N),
          (_O, _N), (_CB, _N), (_C, _CB), (_O, _CB), (_C, _O)]


def _atoms_kernel(x_ref, ridx_ref, out_ref):
    x = x_ref[...]
    nx, ny, nz = x[:, 0:1], x[:, 1:2], x[:, 2:3]
    cax, cay, caz = x[:, 3:4], x[:, 4:5], x[:, 5:6]
    cx, cy, cz = x[:, 6:7], x[:, 7:8], x[:, 8:9]
    bx, by, bz = cax - nx, cay - ny, caz - nz
    ccx, ccy, ccz = cx - cax, cy - cay, cz - caz
    ax = by * ccz - bz * ccy
    ay = bz * ccx - bx * ccz
    az = bx * ccy - by * ccx
    cbx = -0.58273431 * ax + 0.56802827 * bx - 0.54067466 * ccx + cax
    cby = -0.58273431 * ay + 0.56802827 * by - 0.54067466 * ccy + cay
    cbz = -0.58273431 * az + 0.56802827 * bz - 0.54067466 * ccz + caz
    out_ref[:, 0:12] = x
    out_ref[:, 12:13] = cbx
    out_ref[:, 13:14] = cby
    out_ref[:, 14:15] = cbz
    out_ref[:, 15:16] = ridx_ref[...]


def _topk_kernel(carows_ref, caT_ref, mrow_ref, mcol_ref, out_ref):
    caT = caT_ref[0]
    cr = carows_ref[...]
    d2 = None
    for c in range(3):
        diff = cr[:, c:c + 1] - caT[c:c + 1, :]
        sq = diff * diff
        d2 = sq if d2 is None else d2 + sq
    m2 = mrow_ref[...] * mcol_ref[0]
    dist = m2 * jnp.sqrt(d2 + 1e-6)
    dmax = jnp.max(dist, axis=1, keepdims=True)
    vals = dist + (1.0 - m2) * dmax
    iota = lax.broadcasted_iota(jnp.int32, vals.shape, 1).astype(jnp.float32)
    L = vals.shape[1]
    cols = []
    for _ in range(TOP_K):
        mn = jnp.min(vals, axis=1, keepdims=True)
        sel = vals <= mn
        idxf = jnp.min(jnp.where(sel, iota, float(2 * L)), axis=1,
                       keepdims=True)
        cols.append(idxf)
        vals = jnp.where(iota == idxf, 3.0e38, vals)
    idx_all = jnp.concatenate(cols, axis=1)
    out_ref[...] = idx_all.astype(jnp.int32) + pl.program_id(0) * L


def _sc_gather(table, idx, D):
    Bn = idx.shape[0]
    info = plsc.get_sparse_core_info()
    NW = info.num_cores * info.num_subcores
    b_per_w = Bn // NW

    @functools.partial(
        pl.kernel,
        mesh=plsc.VectorSubcoreMesh(core_axis_name="c", subcore_axis_name="s"),
        compiler_params=pltpu.CompilerParams(use_tc_tiling_on_sc=False),
        out_type=jax.ShapeDtypeStruct((Bn, D), jnp.float32),
        scratch_types=[
            pltpu.VMEM((b_per_w,), jnp.int32),
            pltpu.VMEM((b_per_w, D), jnp.float32),
            pltpu.SemaphoreType.DMA,
        ],
    )
    def k(table_hbm, idx_hbm, out_hbm, idx_v, rows_v, sem):
        wid = lax.axis_index("s") * info.num_cores + lax.axis_index("c")
        base = wid * b_per_w
        pltpu.sync_copy(idx_hbm.at[pl.ds(base, b_per_w)], idx_v)
        pltpu.async_copy(table_hbm.at[idx_v], rows_v, sem).wait()
        pltpu.sync_copy(rows_v, out_hbm.at[pl.ds(base, b_per_w)])

    return k(table, idx)


def _edge_kernel(nb_ref, rows_ref, wpeT_ref, bpe_ref, sa_ref, sb_ref,
                 gs_ref, ex_ref, mu400_ref, wedgeT_ref, gam_ref, bet_ref,
                 out_ref):
    T = rows_ref.shape[0]
    E = nb_ref.shape[0]
    nb = nb_ref[...]
    ie = lax.broadcasted_iota(jnp.int32, (E, T), 0)
    ir = lax.broadcasted_iota(jnp.int32, (E, T), 1) * TOP_K
    rep = jnp.where((ie >= ir) & (ie < ir + TOP_K), 1.0, 0.0)
    center = jnp.dot(rep, rows_ref[...],
                     preferred_element_type=jnp.float32,
                     precision=lax.Precision.HIGHEST)
    dcls = jnp.clip(center[:, _RIDX:_RIDX + 1] - nb[:, _RIDX:_RIDX + 1]
                    + float(MAX_REL), 0.0, float(2 * MAX_REL))
    iota66 = lax.broadcasted_iota(jnp.int32, (E, 2 * MAX_REL + 2),
                                  1).astype(jnp.float32)
    oh66 = jnp.where((iota66 > dcls - 0.5) & (iota66 < dcls + 0.5), 1.0, 0.0)
    pos = jnp.dot(oh66, wpeT_ref[...],
                  preferred_element_type=jnp.float32,
                  precision=lax.Precision.HIGHEST) + bpe_ref[...]
    sigma = (22.0 - 2.0) / NUM_RBF
    hp = lambda a, b: jnp.dot(a, b, preferred_element_type=jnp.float32,
                              precision=lax.Precision.HIGHEST)
    ca75 = hp(center, sa_ref[...])
    nb75 = hp(nb, sb_ref[...])
    diff = ca75 - nb75
    d2 = hp(diff * diff, gs_ref[...])
    dist = jnp.sqrt(d2 + 1e-6)
    d400 = hp(dist, ex_ref[...])
    z = (d400 - mu400_ref[...]) / sigma
    rbf = jnp.exp(-(z * z))
    feats = jnp.concatenate([pos, rbf], axis=1)
    e_out = jnp.dot(feats, wedgeT_ref[...],
                    preferred_element_type=jnp.float32)
    m = jnp.mean(e_out, axis=1, keepdims=True)
    var = jnp.mean((e_out - m) ** 2, axis=1, keepdims=True)
    out_ref[...] = ((e_out - m) / jnp.sqrt(var + 1e-5)
                    * gam_ref[...] + bet_ref[...])


def kernel(X, mask, residue_idx, W_pe, b_pe, W_edge, ln_gamma, ln_beta):
    B, L = X.shape[0], X.shape[1]
    K = min(TOP_K, L)
    BL = B * L
    EF = W_edge.shape[0]

    x_flat = X.reshape(BL, 12)
    ridx_f = residue_idx.astype(jnp.float32).reshape(BL, 1)
    atoms = pl.pallas_call(
        _atoms_kernel,
        out_shape=jax.ShapeDtypeStruct((BL, 16), jnp.float32),
    )(x_flat, ridx_f)

    TT = 128
    nblk = L // TT
    ca_rows = X[:, :, 1, :].reshape(BL, 3)
    caT = jnp.transpose(X[:, :, 1, :], (0, 2, 1))
    mask_row = mask.reshape(BL, 1)
    mask_col = mask.reshape(B, 1, L)
    eidx = pl.pallas_call(
        _topk_kernel,
        grid=(B, nblk),
        in_specs=[
            pl.BlockSpec((TT, 3), lambda b, t: (b * nblk + t, 0)),
            pl.BlockSpec((1, 3, L), lambda b, t: (b, 0, 0)),
            pl.BlockSpec((TT, 1), lambda b, t: (b * nblk + t, 0)),
            pl.BlockSpec((1, 1, L), lambda b, t: (b, 0, 0)),
        ],
        out_specs=pl.BlockSpec((TT, K), lambda b, t: (b * nblk + t, 0)),
        out_shape=jax.ShapeDtypeStruct((BL, K), jnp.int32),
    )(ca_rows, caT, mask_row, mask_col)

    nb_flat = _sc_gather(atoms, eidx.reshape(BL * K), 16)

    T = 16
    EB = T * K
    nblk2 = L // T
    wpeT = W_pe.T
    bpe2 = b_pe.reshape(1, -1)
    npair = len(_PAIRS)
    sa = np.zeros((16, 3 * npair), np.float32)
    sb = np.zeros((16, 3 * npair), np.float32)
    gs = np.zeros((3 * npair, npair), np.float32)
    ex = np.zeros((npair, npair * NUM_RBF), np.float32)
    for p, (ao, bo) in enumerate(_PAIRS):
        for c in range(3):
            sa[ao + c, 3 * p + c] = 1.0
            sb[bo + c, 3 * p + c] = 1.0
            gs[3 * p + c, p] = 1.0
        ex[p, p * NUM_RBF:(p + 1) * NUM_RBF] = 1.0
    sa, sb, gs, ex = map(jnp.asarray, (sa, sb, gs, ex))
    mu400 = jnp.tile(jnp.linspace(2.0, 22.0, NUM_RBF),
                     npair).reshape(1, npair * NUM_RBF)
    wedgeT = W_edge.T
    gam2 = ln_gamma.reshape(1, -1)
    bet2 = ln_beta.reshape(1, -1)
    full = lambda s: pl.BlockSpec(s, lambda b, t: tuple(0 for _ in s))
    e_flat = pl.pallas_call(
        _edge_kernel,
        grid=(B, nblk2),
        in_specs=[
            pl.BlockSpec((EB, 16), lambda b, t: (b * nblk2 + t, 0)),
            pl.BlockSpec((T, 16), lambda b, t: (b * nblk2 + t, 0)),
            full(wpeT.shape),
            full(bpe2.shape),
            full(sa.shape),
            full(sb.shape),
            full(gs.shape),
            full(ex.shape),
            full(mu400.shape),
            full(wedgeT.shape),
            full(gam2.shape),
            full(bet2.shape),
        ],
        out_specs=pl.BlockSpec((EB, EF), lambda b, t: (b * nblk2 + t, 0)),
        out_shape=jax.ShapeDtypeStruct((BL * K, EF), jnp.float32),
    )(nb_flat, atoms, wpeT, bpe2, sa, sb, gs, ex, mu400,
      wedgeT, gam2, bet2)
    return e_flat.reshape(B, L, K, EF)
